# baseline (speedup 1.0000x reference)
import functools

import jax
import jax.numpy as jnp
from jax import lax
from jax.experimental import pallas as pl
from jax.experimental.pallas import tpu as pltpu
from jax.experimental.pallas import tpu_sc as plsc

_BATCH = 4
_NUM_CORES = 2
_NUM_SUBCORES = 16
_NUM_WORKERS = _NUM_CORES * _NUM_SUBCORES
_TC_ROWS = 4096
_TC_BS = 1024
_SC_CHUNK = 64


def _tc_bcast_kernel(pe_ref, out_ref):
    blk = pe_ref[...]
    out_ref[...] = jnp.broadcast_to(blk[None], (_BATCH,) + blk.shape)


def _tc_part(pe_weight, batch, d_model):
    return pl.pallas_call(
        _tc_bcast_kernel,
        grid=(_TC_ROWS // _TC_BS,),
        in_specs=[pl.BlockSpec((_TC_BS, d_model), lambda i: (i, 0))],
        out_specs=pl.BlockSpec((batch, _TC_BS, d_model), lambda i: (0, i, 0)),
        out_shape=jax.ShapeDtypeStruct((batch, _TC_ROWS, d_model), pe_weight.dtype),
    )(pe_weight)


def _sc_part(pe_weight, batch, max_len, d_model):
    sc_rows = max_len - _TC_ROWS
    rows_per_worker = sc_rows // _NUM_WORKERS
    assert rows_per_worker % _SC_CHUNK == 0
    n_chunks = rows_per_worker // _SC_CHUNK
    n_buf = 2

    mesh = plsc.VectorSubcoreMesh(core_axis_name="c", subcore_axis_name="s")

    @functools.partial(
        pl.kernel,
        mesh=mesh,
        out_type=jax.ShapeDtypeStruct((batch, sc_rows, d_model), pe_weight.dtype),
        scratch_types=(
            [pltpu.VMEM((_SC_CHUNK, d_model), pe_weight.dtype)] * n_buf
            + [pltpu.SemaphoreType.DMA] * n_buf
            + [pltpu.SemaphoreType.DMA] * n_buf
        ),
    )
    def _sc_bcast(pe_hbm, out_hbm, *scratch):
        bufs = scratch[:n_buf]
        in_sems = scratch[n_buf : 2 * n_buf]
        out_sems = scratch[2 * n_buf :]
        wid = lax.axis_index("s") * _NUM_CORES + lax.axis_index("c")
        base = wid * rows_per_worker

        def in_copy(i):
            row = base + i * _SC_CHUNK
            return pltpu.make_async_copy(
                pe_hbm.at[pl.ds(_TC_ROWS + row, _SC_CHUNK)],
                bufs[i % n_buf],
                in_sems[i % n_buf],
            )

        def out_copies(i):
            row = base + i * _SC_CHUNK
            return [
                pltpu.make_async_copy(
                    bufs[i % n_buf],
                    out_hbm.at[b, pl.ds(row, _SC_CHUNK)],
                    out_sems[i % n_buf],
                )
                for b in range(_BATCH)
            ]

        for i in range(min(n_buf, n_chunks)):
            in_copy(i).start()
        for i in range(n_chunks):
            if i >= n_buf:
                for c in out_copies(i - n_buf):
                    c.wait()
                in_copy(i).start()
            in_copy(i).wait()
            for c in out_copies(i):
                c.start()
        for i in range(max(0, n_chunks - n_buf), n_chunks):
            for c in out_copies(i):
                c.wait()

    return _sc_bcast(pe_weight)


def kernel(x, pe_weight):
    batch = x.shape[0]
    max_len, d_model = pe_weight.shape
    assert batch == _BATCH
    tc_out = _tc_part(pe_weight, batch, d_model)
    sc_out = _sc_part(pe_weight, batch, max_len, d_model)
    return jnp.concatenate([tc_out, sc_out], axis=1)


# speedup vs baseline: 1.6532x; 1.6532x over previous
import functools

import jax
import jax.numpy as jnp
from jax import lax
from jax.experimental import pallas as pl
from jax.experimental.pallas import tpu as pltpu
from jax.experimental.pallas import tpu_sc as plsc

_BATCH = 4
_NUM_CORES = 2
_NUM_SUBCORES = 16
_NUM_WORKERS = _NUM_CORES * _NUM_SUBCORES
_CHUNK = 128


def kernel(x, pe_weight):
    batch = x.shape[0]
    max_len, d_model = pe_weight.shape
    assert batch == _BATCH and max_len % _NUM_WORKERS == 0
    rows_per_worker = max_len // _NUM_WORKERS
    assert rows_per_worker % _CHUNK == 0
    n_chunks = rows_per_worker // _CHUNK

    mesh = plsc.VectorSubcoreMesh(core_axis_name="c", subcore_axis_name="s")

    @functools.partial(
        pl.kernel,
        mesh=mesh,
        out_type=jax.ShapeDtypeStruct((batch, max_len, d_model), pe_weight.dtype),
        scratch_types=[
            pltpu.VMEM_SHARED((_NUM_SUBCORES * _CHUNK, d_model), pe_weight.dtype),
            pltpu.SemaphoreType.DMA,
        ],
    )
    def _sc_bcast(pe_hbm, out_hbm, shared, sem):
        sid = lax.axis_index("s")
        wid = sid * _NUM_CORES + lax.axis_index("c")
        base = wid * rows_per_worker
        my_stage = shared.at[pl.ds(sid * _CHUNK, _CHUNK)]

        def body(i, _):
            row = base + i * _CHUNK
            pltpu.sync_copy(pe_hbm.at[pl.ds(row, _CHUNK)], my_stage)
            copies = [
                pltpu.make_async_copy(
                    my_stage, out_hbm.at[b, pl.ds(row, _CHUNK)], sem
                )
                for b in range(_BATCH)
            ]
            for c in copies:
                c.start()
            for c in copies:
                c.wait()
            return ()

        lax.fori_loop(0, n_chunks, body, ())

    return _sc_bcast(pe_weight)


# speedup vs baseline: 2.8600x; 1.7300x over previous
import jax
import jax.numpy as jnp
from jax.experimental import pallas as pl
from jax.experimental.pallas import tpu as pltpu

_BATCH = 4
_BS = 1024


def _bcast_kernel(pe_ref, out_ref, sems):
    i = pl.program_id(0)
    copies = [
        pltpu.make_async_copy(
            pe_ref, out_ref.at[b, pl.ds(i * _BS, _BS)], sems.at[b]
        )
        for b in range(_BATCH)
    ]
    for c in copies:
        c.start()
    for c in copies:
        c.wait()


def kernel(x, pe_weight):
    batch = x.shape[0]
    max_len, d_model = pe_weight.shape
    assert batch == _BATCH and max_len % _BS == 0
    return pl.pallas_call(
        _bcast_kernel,
        grid=(max_len // _BS,),
        in_specs=[pl.BlockSpec((_BS, d_model), lambda i: (i, 0))],
        out_specs=pl.BlockSpec(memory_space=pl.ANY),
        out_shape=jax.ShapeDtypeStruct((batch, max_len, d_model), pe_weight.dtype),
        scratch_shapes=[pltpu.SemaphoreType.DMA((_BATCH,))],
    )(pe_weight)
